# Initial kernel scaffold; baseline (speedup 1.0000x reference)
import jax
import jax.numpy as jnp
from jax.experimental import pallas as pl


def kernel(xyz, T_xy, T_yz, T_zx):
    raise NotImplementedError("write your pallas kernel here")


# speedup vs baseline: 64.9674x; 64.9674x over previous
import functools

import jax
import jax.numpy as jnp
from jax import lax
from jax.experimental import pallas as pl
from jax.experimental.pallas import tpu as pltpu
from jax.experimental.pallas import tpu_sc as plsc

C = 32
W = 512
HW = W * W
NPLANES = 3
NPTS = 524288
NTAP = 12

_info = plsc.get_sparse_core_info()
NC, NS, L = _info.num_cores, _info.num_subcores, _info.num_lanes
NW = NC * NS
PPW = NPTS // NW
B = 128
NB = PPW // B
G = B // L


def _tap_setup(u, v, plane):
    iu = u * (W * 0.5) + (W - 1) * 0.5
    iv = v * (W * 0.5) + (W - 1) * 0.5
    iu0 = iu.astype(jnp.int32)
    iv0 = iv.astype(jnp.int32)
    fu = iu - iu0.astype(jnp.float32)
    fv = iv - iv0.astype(jnp.float32)
    u1ok = iu0 < (W - 1)
    v1ok = iv0 < (W - 1)
    iu1 = jnp.where(u1ok, iu0 + 1, W - 1)
    iv1 = jnp.where(v1ok, iv0 + 1, W - 1)
    wu1 = jnp.where(u1ok, fu, 0.0)
    wv1 = jnp.where(v1ok, fv, 0.0)
    wu0 = 1.0 - fu
    wv0 = 1.0 - fv
    r0 = iv0 * W + plane * HW
    r1 = iv1 * W + plane * HW
    rows = (r0 + iu0, r0 + iu1, r1 + iu0, r1 + iu1)
    wts = (wu0 * wv0, wu1 * wv0, wu0 * wv1, wu1 * wv1)
    return rows, wts


@functools.partial(
    pl.kernel,
    mesh=plsc.VectorSubcoreMesh(core_axis_name="c", subcore_axis_name="s"),
    out_type=jax.ShapeDtypeStruct((NPTS, C), jnp.float32),
    compiler_params=pltpu.CompilerParams(use_tc_tiling_on_sc=False),
    scratch_types=[
        pltpu.VMEM((NPLANES, B), jnp.float32),
        pltpu.VMEM((NTAP, B), jnp.int32),
        pltpu.VMEM((NTAP, B), jnp.float32),
        pltpu.VMEM((NTAP, B, C), jnp.float32),
        pltpu.VMEM((B, C), jnp.float32),
        pltpu.SemaphoreType.DMA,
    ],
)
def _tri_gather(xyz_hbm, tab_hbm, out_hbm, uvw_v, idx_v, w_v, rows_v, out_v, sem):
    wid = lax.axis_index("s") * NC + lax.axis_index("c")
    base = wid * PPW
    iota = lax.iota(jnp.int32, L)

    def batch_body(j, carry):
        row0 = base + j * B
        pltpu.sync_copy(xyz_hbm.at[:, pl.ds(row0, B)], uvw_v)

        for g in range(G):
            sl = pl.ds(g * L, L)
            coords = (uvw_v[0, sl], uvw_v[1, sl], uvw_v[2, sl])
            for p in range(NPLANES):
                rows, wts = _tap_setup(coords[p], coords[(p + 1) % 3], p)
                for t in range(4):
                    idx_v[4 * p + t, sl] = rows[t]
                    w_v[4 * p + t, sl] = wts[t]

        cps = [pltpu.async_copy(tab_hbm.at[idx_v.at[t]], rows_v.at[t], sem)
               for t in range(NTAP)]
        for cp in cps:
            cp.wait()

        def group_body(g2, c2):
            base_pp = g2 * L
            wvecs = [w_v[t, pl.ds(base_pp, L)] for t in range(NTAP)]
            for lane in range(L):
                pp = base_pp + lane
                acc0 = jnp.zeros((L,), jnp.float32)
                acc1 = jnp.zeros((L,), jnp.float32)
                for t in range(NTAP):
                    wt = jnp.full((L,), wvecs[t][lane], jnp.float32)
                    acc0 = acc0 + wt * rows_v[t, pp, pl.ds(0, L)]
                    acc1 = acc1 + wt * rows_v[t, pp, pl.ds(L, L)]
                out_v[pp, pl.ds(0, L)] = acc0
                out_v[pp, pl.ds(L, L)] = acc1
            return c2

        lax.fori_loop(0, G, group_body, 0)
        pltpu.sync_copy(out_v, out_hbm.at[pl.ds(row0, B)])
        return carry

    lax.fori_loop(0, NB, batch_body, 0)


def kernel(xyz, T_xy, T_yz, T_zx):
    tab = jnp.concatenate(
        [jnp.transpose(T[0], (1, 2, 0)).reshape(HW, C) for T in (T_xy, T_yz, T_zx)],
        axis=0)
    xyz_t = xyz.T
    return _tri_gather(xyz_t, tab)


# speedup vs baseline: 84.1741x; 1.2956x over previous
import functools

import jax
import jax.numpy as jnp
from jax import lax
from jax.experimental import pallas as pl
from jax.experimental.pallas import tpu as pltpu
from jax.experimental.pallas import tpu_sc as plsc

C = 32
CP = C // 2
W = 512
HW = W * W
NPLANES = 3
NPTS = 524288
NTAP = 12

_info = plsc.get_sparse_core_info()
NC, NS, L = _info.num_cores, _info.num_subcores, _info.num_lanes
NW = NC * NS
PPW = NPTS // NW
B = 128
NB = PPW // B
G = B // L


def _tap_setup(u, v, plane):
    iu = u * (W * 0.5) + (W - 1) * 0.5
    iv = v * (W * 0.5) + (W - 1) * 0.5
    iu0 = iu.astype(jnp.int32)
    iv0 = iv.astype(jnp.int32)
    fu = iu - iu0.astype(jnp.float32)
    fv = iv - iv0.astype(jnp.float32)
    u1ok = iu0 < (W - 1)
    v1ok = iv0 < (W - 1)
    iu1 = jnp.where(u1ok, iu0 + 1, W - 1)
    iv1 = jnp.where(v1ok, iv0 + 1, W - 1)
    wu1 = jnp.where(u1ok, fu, 0.0)
    wv1 = jnp.where(v1ok, fv, 0.0)
    wu0 = 1.0 - fu
    wv0 = 1.0 - fv
    r0 = iv0 * W + plane * HW
    r1 = iv1 * W + plane * HW
    rows = (r0 + iu0, r0 + iu1, r1 + iu0, r1 + iu1)
    wts = (wu0 * wv0, wu1 * wv0, wu0 * wv1, wu1 * wv1)
    return rows, wts


@functools.partial(
    pl.kernel,
    mesh=plsc.VectorSubcoreMesh(core_axis_name="c", subcore_axis_name="s"),
    out_type=jax.ShapeDtypeStruct((NPTS, C), jnp.float32),
    compiler_params=pltpu.CompilerParams(
        use_tc_tiling_on_sc=False, needs_layout_passes=False),
    scratch_types=[
        pltpu.VMEM((NPLANES, B), jnp.float32),
        pltpu.VMEM((2, NTAP, B), jnp.int32),
        pltpu.VMEM((2, NTAP, B), jnp.int32),
        pltpu.VMEM((2, NTAP, B, CP), jnp.int32),
        pltpu.VMEM((B, C), jnp.float32),
        pltpu.SemaphoreType.DMA,
    ],
)
def _tri_gather(xyz_hbm, tab_hbm, out_hbm, uvw_v, idx_v, w_v, rows_v, out_v, sem):
    wid = lax.axis_index("s") * NC + lax.axis_index("c")
    base = wid * PPW
    iota = lax.iota(jnp.int32, L)

    def build(j, d):
        row0 = base + j * B
        pltpu.sync_copy(xyz_hbm.at[:, pl.ds(row0, B)], uvw_v)
        for g in range(G):
            sl = pl.ds(g * L, L)
            coords = (uvw_v[0, sl], uvw_v[1, sl], uvw_v[2, sl])
            for p in range(NPLANES):
                rows, wts = _tap_setup(coords[p], coords[(p + 1) % 3], p)
                for t in range(4):
                    idx_v[d, 4 * p + t, sl] = rows[t]
                    w_v[d, 4 * p + t, sl] = plsc.bitcast(
                        plsc.pack(wts[t], wts[t],
                                  format=plsc.PackFormat.INTERLEAVED),
                        jnp.int32)

    def fire(d):
        for t in range(NTAP):
            pltpu.async_copy(tab_hbm.at[idx_v.at[d, t]], rows_v.at[d, t], sem)

    def drain(d):
        for t in range(NTAP):
            pltpu.make_async_copy(
                tab_hbm.at[idx_v.at[d, t]], rows_v.at[d, t], sem).wait()

    def accumulate(j, d):
        def group_body(g, c2):
            base_pp = g * L
            wwords = [w_v[d, t, pl.ds(base_pp, L)] for t in range(NTAP)]
            for lane in range(L):
                pp = base_pp + lane
                acc_a = jnp.zeros((L,), jnp.float32)
                acc_b = jnp.zeros((L,), jnp.float32)
                for p in range(NPLANES):
                    pacc = None
                    for t in range(4 * p, 4 * p + 4):
                        wsplat = plsc.bitcast(
                            jnp.full((L,), wwords[t][lane], jnp.int32),
                            jnp.bfloat16)
                        row = plsc.bitcast(rows_v[d, t, pp, :], jnp.bfloat16)
                        term = wsplat * row
                        pacc = term if pacc is None else pacc + term
                    a, b = plsc.unpack(
                        pacc, format=plsc.PackFormat.INTERLEAVED,
                        preferred_element_type=jnp.float32)
                    acc_a = acc_a + a
                    acc_b = acc_b + b
                out_v[pp, pl.ds(0, L)] = acc_a
                out_v[pp, pl.ds(L, L)] = acc_b
            return c2

        lax.fori_loop(0, G, group_body, 0)
        pltpu.sync_copy(out_v, out_hbm.at[pl.ds(base + j * B, B)])

    build(0, 0)
    fire(0)

    def batch_body(j, carry):
        d = lax.bitwise_and(j, 1)
        build(j + 1, 1 - d)
        drain(d)
        fire(1 - d)
        accumulate(j, d)
        return carry

    lax.fori_loop(0, NB - 1, batch_body, 0)
    drain(lax.bitwise_and(NB - 1, 1))
    accumulate(NB - 1, lax.bitwise_and(NB - 1, 1))


def kernel(xyz, T_xy, T_yz, T_zx):
    def prep(T):
        t = jnp.transpose(T[0], (1, 2, 0)).astype(jnp.bfloat16)
        pairs = jnp.stack([t[..., :CP], t[..., CP:]], axis=-1)
        return lax.bitcast_convert_type(pairs, jnp.int32).reshape(HW, CP)

    tab = jnp.concatenate([prep(T) for T in (T_xy, T_yz, T_zx)], axis=0)
    xyz_t = xyz.T
    return _tri_gather(xyz_t, tab)
